# Initial kernel scaffold; baseline (speedup 1.0000x reference)
import jax
import jax.numpy as jnp
from jax.experimental import pallas as pl


def kernel(predicts, target):
    raise NotImplementedError("write your pallas kernel here")


# speedup vs baseline: 8.4348x; 8.4348x over previous
import functools
import math

import jax
import jax.numpy as jnp
from jax.experimental import pallas as pl
from jax.experimental.pallas import tpu as pltpu

PAD = 0
SMOOTH = 0.1
CONF = 1.0 - SMOOTH

ROWS_BLK = 128


def _loss_block(pred_ref, tgt_ref, out_ref):
    i = pl.program_id(0)

    x = pred_ref[...]
    t = tgt_ref[0, 0, :]
    v = x.shape[1]
    eps = SMOOTH / (v - 2)
    k_const = CONF * math.log(CONF) + SMOOTH * math.log(eps)

    row_sum = jnp.sum(x, axis=1)
    col = jax.lax.broadcasted_iota(jnp.int32, x.shape, 1)
    p_t = jnp.sum(jnp.where(col == t[:, None], x, 0.0), axis=1)
    p_0 = x[:, 0]

    valid = (t != PAD)
    per_row = k_const + (eps - CONF) * p_t + eps * p_0 - eps * row_sum
    partial = jnp.sum(jnp.where(valid, per_row, 0.0))

    @pl.when(i == 0)
    def _init():
        out_ref[...] = jnp.zeros((1, 1), jnp.float32)

    out_ref[...] += partial.reshape(1, 1)


@functools.partial(jax.jit, static_argnames=())
def kernel(predicts, target):
    n, v = predicts.shape
    grid = n // ROWS_BLK
    tgt3 = target.astype(jnp.int32).reshape(grid, 1, ROWS_BLK)

    out = pl.pallas_call(
        _loss_block,
        grid=(grid,),
        in_specs=[
            pl.BlockSpec((ROWS_BLK, v), lambda i: (i, 0)),
            pl.BlockSpec((1, 1, ROWS_BLK), lambda i: (i, 0, 0)),
        ],
        out_specs=pl.BlockSpec((1, 1), lambda i: (0, 0)),
        out_shape=jax.ShapeDtypeStruct((1, 1), jnp.float32),
    )(predicts, tgt3)
    return out[0, 0]


# speedup vs baseline: 8.7062x; 1.0322x over previous
import functools
import math

import jax
import jax.numpy as jnp
from jax.experimental import pallas as pl
from jax.experimental.pallas import tpu as pltpu

PAD = 0
SMOOTH = 0.1
CONF = 1.0 - SMOOTH

ROWS_BLK = 128


def _loss_block(pred_ref, tgt_ref, out_ref):
    i = pl.program_id(0)

    x = pred_ref[...]
    t = tgt_ref[0, 0, :]
    v = x.shape[1]
    eps = SMOOTH / (v - 2)
    k_const = CONF * math.log(CONF) + SMOOTH * math.log(eps)

    ratio = CONF / eps
    col = jax.lax.broadcasted_iota(jnp.int32, x.shape, 1)
    z = jnp.where(col == t[:, None], x * ratio, x)
    row_acc = jnp.sum(z, axis=1)
    p_0 = x[:, 0]

    valid = (t != PAD)
    per_row = k_const + eps * p_0 - eps * row_acc
    partial = jnp.sum(jnp.where(valid, per_row, 0.0))

    @pl.when(i == 0)
    def _init():
        out_ref[...] = jnp.zeros((1, 1), jnp.float32)

    out_ref[...] += partial.reshape(1, 1)


@functools.partial(jax.jit, static_argnames=())
def kernel(predicts, target):
    n, v = predicts.shape
    grid = n // ROWS_BLK
    tgt3 = target.astype(jnp.int32).reshape(grid, 1, ROWS_BLK)

    out = pl.pallas_call(
        _loss_block,
        grid=(grid,),
        in_specs=[
            pl.BlockSpec((ROWS_BLK, v), lambda i: (i, 0)),
            pl.BlockSpec((1, 1, ROWS_BLK), lambda i: (i, 0, 0)),
        ],
        out_specs=pl.BlockSpec((1, 1), lambda i: (0, 0)),
        out_shape=jax.ShapeDtypeStruct((1, 1), jnp.float32),
    )(predicts, tgt3)
    return out[0, 0]
